# Initial kernel scaffold; baseline (speedup 1.0000x reference)
import jax
import jax.numpy as jnp
from jax.experimental import pallas as pl


def kernel(variate_ids, value_num, cat_ids, variate_type, numeric_means, numeric_stds, W1, b1, W2, b2, cat_table, ln_gamma, ln_beta):
    raise NotImplementedError("write your pallas kernel here")


# speedup vs baseline: 10.6664x; 10.6664x over previous
import functools

import jax
import jax.numpy as jnp
from jax import lax
from jax.experimental import pallas as pl
from jax.experimental.pallas import tpu as pltpu
from jax.experimental.pallas import tpu_sc as plsc

_CHUNK = 128


def _make_sc_gather(n_tokens: int, d: int, table_rows: int):
    info = plsc.get_sparse_core_info()
    nw = info.num_cores * info.num_subcores
    chunk_rows = n_tokens // _CHUNK
    rows_per_w = chunk_rows // nw
    per_w = rows_per_w * _CHUNK

    mesh = plsc.VectorSubcoreMesh(core_axis_name="c", subcore_axis_name="s")

    @functools.partial(
        pl.kernel,
        mesh=mesh,
        out_type=jax.ShapeDtypeStruct((n_tokens, d), jnp.float32),
        scratch_types=[
            pltpu.VMEM((rows_per_w, _CHUNK), jnp.int32),
            pltpu.VMEM((_CHUNK, d), jnp.float32),
            pltpu.SemaphoreType.DMA,
        ],
    )
    def gather_kernel(idx_hbm, table_hbm, out_hbm, idx_v, rows_v, sem):
        wid = lax.axis_index("s") * info.num_cores + lax.axis_index("c")
        tok_base = wid * per_w
        pltpu.sync_copy(idx_hbm.at[wid], idx_v)

        def body(j, carry):
            pltpu.async_copy(table_hbm.at[idx_v.at[j]], rows_v, sem).wait()
            pltpu.sync_copy(rows_v, out_hbm.at[pl.ds(tok_base + j * _CHUNK, _CHUNK)])
            return carry

        lax.fori_loop(0, rows_per_w, body, 0)

    return gather_kernel


def _tc_body(p_ref, g_ref, tabs_ref, w1_ref, b1_ref, w2_ref, b2_ref,
             gam_ref, bet_ref, o_ref):
    p = p_ref[...]
    vidf = p[:, 0:1]
    catf = p[:, 1:2]
    val = p[:, 2:3]
    iot = lax.broadcasted_iota(jnp.int32, (1, tabs_ref.shape[0]), 1).astype(jnp.float32)
    onehot = (vidf == iot).astype(jnp.float32)
    scal = jnp.dot(onehot, tabs_ref[...],
                   preferred_element_type=jnp.float32)
    vt = scal[:, 0:1]
    mu = scal[:, 1:2]
    sg = scal[:, 2:3]
    mask_num = vt == 0.0
    mask_cat = jnp.logical_and(vt == 1.0, catf >= 0.0)
    v = (val - mu) / (sg + 1e-12)
    h = jnp.maximum(v * w1_ref[...] + b1_ref[...], 0.0)
    e_num = jnp.dot(h, w2_ref[...],
                    preferred_element_type=jnp.float32) + b2_ref[...]
    e = jnp.where(mask_num, e_num, 0.0)
    e = jnp.where(mask_cat, g_ref[...], e)
    m = jnp.mean(e, axis=-1, keepdims=True)
    dev = e - m
    var = jnp.mean(dev * dev, axis=-1, keepdims=True)
    o_ref[...] = dev * lax.rsqrt(var + 1e-5) * gam_ref[...] + bet_ref[...]


def _tc_compute(p, gathered, tabs, w1, b1, w2, b2, gamma, beta, block_r: int):
    n, d = gathered.shape
    grid = (n // block_r,)
    full = lambda shape: pl.BlockSpec(shape, lambda i: (0, 0))
    return pl.pallas_call(
        _tc_body,
        grid=grid,
        in_specs=[
            pl.BlockSpec((block_r, p.shape[1]), lambda i: (i, 0)),
            pl.BlockSpec((block_r, d), lambda i: (i, 0)),
            full(tabs.shape),
            full(w1.shape),
            full(b1.shape),
            full(w2.shape),
            full(b2.shape),
            full(gamma.shape),
            full(beta.shape),
        ],
        out_specs=pl.BlockSpec((block_r, d), lambda i: (i, 0)),
        out_shape=jax.ShapeDtypeStruct((n, d), jnp.float32),
    )(p, gathered, tabs, w1, b1, w2, b2, gamma, beta)


def kernel(variate_ids, value_num, cat_ids, variate_type, numeric_means,
           numeric_stds, W1, b1, W2, b2, cat_table, ln_gamma, ln_beta):
    b, t = variate_ids.shape
    nc, d = cat_table.shape
    n = b * t
    nv = variate_type.shape[0]

    vidf = variate_ids.reshape(n).astype(jnp.float32)
    catf = cat_ids.reshape(n).astype(jnp.float32)
    valf = value_num.reshape(n).astype(jnp.float32)
    zero = jnp.zeros((n,), jnp.float32)
    p = jnp.stack([vidf, catf, valf, zero], axis=-1)

    nvp = 128
    tabs = jnp.zeros((nvp, 8), jnp.float32)
    tabs = tabs.at[:nv, 0].set(variate_type.astype(jnp.float32))
    tabs = tabs.at[:nv, 1].set(numeric_means.astype(jnp.float32))
    tabs = tabs.at[:nv, 2].set(numeric_stds.astype(jnp.float32))

    cat32 = jnp.maximum(cat_ids.reshape(n).astype(jnp.int32), 0)
    info = plsc.get_sparse_core_info()
    nw = info.num_cores * info.num_subcores
    idx3d = cat32.reshape(nw, n // (nw * _CHUNK), _CHUNK)
    gathered = _make_sc_gather(n, d, nc)(idx3d, cat_table)

    out = _tc_compute(
        p, gathered, tabs,
        W1.reshape(1, -1), b1.reshape(1, -1), W2, b2.reshape(1, -1),
        ln_gamma.reshape(1, -1), ln_beta.reshape(1, -1),
        block_r=1024,
    )
    return out.reshape(b, t, d)
